# Initial kernel scaffold; baseline (speedup 1.0000x reference)
import jax
import jax.numpy as jnp
from jax.experimental import pallas as pl


def kernel(img0, img1):
    raise NotImplementedError("write your pallas kernel here")


# speedup vs baseline: 55.3243x; 55.3243x over previous
import functools

import jax
import jax.numpy as jnp
from jax import lax
from jax.experimental import pallas as pl
from jax.experimental.pallas import tpu as pltpu
from jax.experimental.pallas import tpu_sc as plsc

_NBIN = 256
_N = 4096 * 4096
_NWORKERS = 32
_EPW = _N // _NWORKERS
_CHUNK = 16384
_NCHUNK = _EPW // _CHUNK
_LANES = 16
_HROW = 264
_HSIZE = _LANES * _HROW


def _min_body(x_ref, o_ref):
    i = pl.program_id(0)
    m = jnp.min(x_ref[...])

    @pl.when(i == 0)
    def _():
        o_ref[0, 0] = m

    @pl.when(i > 0)
    def _():
        o_ref[0, 0] = jnp.minimum(o_ref[0, 0], m)


def _global_min(img):
    return pl.pallas_call(
        _min_body,
        grid=(16,),
        in_specs=[pl.BlockSpec((256, 4096), lambda i: (i, 0))],
        out_specs=pl.BlockSpec(memory_space=pltpu.SMEM),
        out_shape=jax.ShapeDtypeStruct((1, 1), jnp.float32),
    )(img)


def _hist_worker(img_hbm, out_hbm, buf, hist, hminv, invdh, wid):
    lane_base = jnp.arange(_LANES, dtype=jnp.int32) * _HROW
    zeros16 = jnp.zeros((_LANES,), jnp.float32)

    def zero_body(j, _):
        hist[pl.ds(j * _LANES, _LANES)] = zeros16
        return _

    lax.fori_loop(0, _HSIZE // _LANES, zero_body, None)

    base = wid * _EPW

    def chunk_body(c, _):
        pltpu.sync_copy(img_hbm.at[pl.ds(base + c * _CHUNK, _CHUNK)], buf)

        def vec_body(v, _):
            x = buf[pl.ds(v * _LANES, _LANES)]
            keep = jnp.logical_and(x >= hminv, x <= 0.0)
            t = (x - hminv) * invdh
            tc = jnp.clip(t, 0.0, 255.0)
            i = tc.astype(jnp.int32)
            fr = tc - i.astype(jnp.float32)
            w0 = jnp.where(keep, 1.0 - fr, 0.0)
            w1 = jnp.where(keep, fr, 0.0)
            idx = lane_base + i
            plsc.addupdate_scatter(hist, [idx], w0)
            plsc.addupdate_scatter(hist, [idx + 1], w1)
            return _

        lax.fori_loop(0, _CHUNK // _LANES, vec_body, None, unroll=8)
        return _

    lax.fori_loop(0, _NCHUNK, chunk_body, None)
    pltpu.sync_copy(hist, out_hbm.at[wid])


def _hist_sc_body(img0_hbm, img1_hbm, hmin_hbm, out0_hbm, out1_hbm,
                  buf, hist, hv):
    wid = lax.axis_index("s") * 2 + lax.axis_index("c")
    pltpu.sync_copy(hmin_hbm, hv)
    hminv = hv[...]
    dh = (0.0 - hminv) * (1.0 / (_NBIN - 1))
    invdh = 1.0 / dh
    _hist_worker(img0_hbm, out0_hbm, buf, hist, hminv, invdh, wid)
    _hist_worker(img1_hbm, out1_hbm, buf, hist, hminv, invdh, wid)


def _hist_sc(img0_flat, img1_flat, hmin_arr):
    mesh = plsc.VectorSubcoreMesh(core_axis_name="c", subcore_axis_name="s")
    f = functools.partial(
        pl.kernel,
        mesh=mesh,
        out_type=[
            jax.ShapeDtypeStruct((_NWORKERS, _HSIZE), jnp.float32),
            jax.ShapeDtypeStruct((_NWORKERS, _HSIZE), jnp.float32),
        ],
        scratch_types=[
            pltpu.VMEM((_CHUNK,), jnp.float32),
            pltpu.VMEM((_HSIZE,), jnp.float32),
            pltpu.VMEM((_LANES,), jnp.float32),
        ],
        compiler_params=pltpu.CompilerParams(needs_layout_passes=False),
    )(_hist_sc_body)
    return f(img0_flat, img1_flat, hmin_arr)


def _kl_body(p0_ref, p1_ref, o_ref):
    eps = 1e-10
    h0 = jnp.sum(p0_ref[...], axis=0, keepdims=True)[:, :_NBIN]
    h1 = jnp.sum(p1_ref[...], axis=0, keepdims=True)[:, :_NBIN]
    h0 = (h0 + eps) / (jnp.sum(h0) + eps)
    h1 = (h1 + eps) / (jnp.sum(h1) + eps)
    inp = jnp.log((h1 + eps) / h1)
    tgt = jnp.log((h1 + eps) / h0)
    o_ref[0, 0] = jnp.mean(jnp.exp(tgt) * (tgt - inp))


def _kl(parts0, parts1):
    return pl.pallas_call(
        _kl_body,
        out_specs=pl.BlockSpec(memory_space=pltpu.SMEM),
        out_shape=jax.ShapeDtypeStruct((1, 1), jnp.float32),
    )(parts0, parts1)


def kernel(img0, img1):
    m = _global_min(img0)[0, 0]
    hmin_arr = jnp.full((_LANES,), m, dtype=jnp.float32)
    parts0, parts1 = _hist_sc(img0.reshape(-1), img1.reshape(-1), hmin_arr)
    p0 = parts0.reshape(_NWORKERS * _LANES, _HROW)
    p1 = parts1.reshape(_NWORKERS * _LANES, _HROW)
    return _kl(p0, p1)[0, 0]
